# Initial kernel scaffold; baseline (speedup 1.0000x reference)
import jax
import jax.numpy as jnp
from jax.experimental import pallas as pl


def kernel(x, edge_index, W, b, gamma, beta):
    raise NotImplementedError("write your pallas kernel here")


# speedup vs baseline: 20.8126x; 20.8126x over previous
import functools

import jax
import jax.numpy as jnp
from jax import lax
from jax.experimental import pallas as pl
from jax.experimental.pallas import tpu as pltpu
from jax.experimental.pallas import tpu_sc as plsc

N = 10000
E = 320000
D = 128

NC, NS, L = 2, 16, 16
NW = NC * NS
EPW = E // NW
CHUNK = 80
NCHUNK = EPW // CHUNK
NP = 10240
RPT = NP // NS

_mesh = plsc.VectorSubcoreMesh(core_axis_name="c", subcore_axis_name="s")
_sc_params = pltpu.CompilerParams(needs_layout_passes=False)


@functools.partial(
    pl.kernel,
    out_type=jax.ShapeDtypeStruct((NW * N,), jnp.float32),
    mesh=_mesh,
    scratch_types=[
        pltpu.VMEM((EPW,), jnp.int32),
        pltpu.VMEM((N,), jnp.float32),
    ],
    compiler_params=_sc_params,
)
def _deg_kernel(dst_hbm, out_hbm, idx_v, cnt_v):
    cid = lax.axis_index("c")
    sid = lax.axis_index("s")
    wid = sid * NC + cid
    pltpu.sync_copy(dst_hbm.at[pl.ds(wid * EPW, EPW)], idx_v)

    def zero_body(i, c):
        cnt_v[pl.ds(i * L, L)] = jnp.zeros((L,), jnp.float32)
        return c

    lax.fori_loop(0, N // L, zero_body, 0)

    ones = jnp.ones((L,), jnp.float32)

    def scat_body(i, c):
        idx = idx_v[pl.ds(i * L, L)]
        plsc.addupdate_scatter(cnt_v, [idx], ones)
        return c

    lax.fori_loop(0, EPW // L, scat_body, 0)
    pltpu.sync_copy(cnt_v, out_hbm.at[pl.ds(wid * N, N)])


def _prep_body(x_ref, w_ref, cnt_ref, hs_ref, dinv_ref):
    deg = jnp.sum(cnt_ref[...], axis=0) + 1.0
    dinv = lax.rsqrt(deg)
    h = jnp.dot(x_ref[...], w_ref[...], preferred_element_type=jnp.float32)
    hs_ref[...] = h * dinv[:, None]
    dinv_ref[...] = dinv[:, None]


_prep_kernel = pl.pallas_call(
    _prep_body,
    out_shape=(
        jax.ShapeDtypeStruct((N, D), jnp.float32),
        jax.ShapeDtypeStruct((N, 1), jnp.float32),
    ),
)


@functools.partial(
    pl.kernel,
    out_type=jax.ShapeDtypeStruct((NC, NP, D), jnp.float32),
    mesh=_mesh,
    scratch_types=[
        pltpu.VMEM((CHUNK,), jnp.int32),
        pltpu.VMEM((CHUNK,), jnp.int32),
        pltpu.VMEM((CHUNK, D), jnp.float32),
        pltpu.SemaphoreType.DMA,
        pltpu.VMEM_SHARED((NP, D), jnp.float32),
    ],
    compiler_params=_sc_params,
)
def _agg_kernel(hs_hbm, src_hbm, dst_hbm, acc_hbm,
                src_idx, dst_idx, rows, sem, acc_sh):
    cid = lax.axis_index("c")
    sid = lax.axis_index("s")
    wid = sid * NC + cid

    def zero_rows(i, c):
        rows[i // (D // L), pl.ds((i % (D // L)) * L, L)] = (
            jnp.zeros((L,), jnp.float32))
        return c

    lax.fori_loop(0, CHUNK * (D // L), zero_rows, 0)

    def zero_acc(j, c):
        pltpu.sync_copy(rows, acc_sh.at[pl.ds(sid * RPT + j * CHUNK, CHUNK)])
        return c

    lax.fori_loop(0, RPT // CHUNK, zero_acc, 0)
    plsc.subcore_barrier()

    base = wid * EPW

    def step(c, carry):
        off = base + c * CHUNK
        pltpu.sync_copy(src_hbm.at[pl.ds(off, CHUNK)], src_idx)
        pltpu.sync_copy(dst_hbm.at[pl.ds(off, CHUNK)], dst_idx)
        pltpu.async_copy(hs_hbm.at[src_idx], rows, sem).wait()
        pltpu.sync_copy(rows, acc_sh.at[dst_idx], add=True)
        return carry

    lax.fori_loop(0, NCHUNK, step, 0)
    plsc.subcore_barrier()
    pltpu.sync_copy(acc_sh.at[pl.ds(sid * RPT, RPT)],
                    acc_hbm.at[cid, pl.ds(sid * RPT, RPT)])


def _post_body(acc_ref, hs_ref, dinv_ref, b_ref, g_ref, be_ref, out_ref):
    a0 = acc_ref[0, pl.ds(0, N), :]
    a1 = acc_ref[1, pl.ds(0, N), :]
    pre = (a0 + a1 + hs_ref[...]) * dinv_ref[...] + b_ref[...]
    pre = jnp.where(pre >= 0.0, pre, 0.01 * pre)
    mean = jnp.mean(pre, axis=0, keepdims=True)
    var = jnp.mean((pre - mean) ** 2, axis=0, keepdims=True)
    out_ref[...] = (pre - mean) * lax.rsqrt(var + 1e-5) * g_ref[...] + be_ref[...]


_post_kernel = pl.pallas_call(
    _post_body,
    out_shape=jax.ShapeDtypeStruct((N, D), jnp.float32),
)


def kernel(x, edge_index, W, b, gamma, beta):
    src = edge_index[0]
    dst = edge_index[1]
    cnt = _deg_kernel(dst).reshape(NW, N)
    hs, dinv = _prep_kernel(x, W, cnt)
    acc = _agg_kernel(hs, src, dst)
    return _post_kernel(acc, hs, dinv,
                        b.reshape(1, D), gamma.reshape(1, D),
                        beta.reshape(1, D))


# speedup vs baseline: 40.4829x; 1.9451x over previous
import functools

import jax
import jax.numpy as jnp
from jax import lax
from jax.experimental import pallas as pl
from jax.experimental.pallas import tpu as pltpu
from jax.experimental.pallas import tpu_sc as plsc

N = 10000
E = 320000
D = 128

NC, NS, L = 2, 16, 16
NW = NC * NS
EPW = E // NW
CHUNK = 80
NCHUNK = EPW // CHUNK
NP = 10240
RPT = NP // NS
SB = 25
NSB = NCHUNK // SB

_mesh = plsc.VectorSubcoreMesh(core_axis_name="c", subcore_axis_name="s")
_sc_params = pltpu.CompilerParams(needs_layout_passes=False)


@functools.partial(
    pl.kernel,
    out_type=jax.ShapeDtypeStruct((NW * N,), jnp.float32),
    mesh=_mesh,
    scratch_types=[
        pltpu.VMEM((EPW,), jnp.int32),
        pltpu.VMEM((N,), jnp.float32),
    ],
    compiler_params=_sc_params,
)
def _deg_kernel(dst_hbm, out_hbm, idx_v, cnt_v):
    cid = lax.axis_index("c")
    sid = lax.axis_index("s")
    wid = sid * NC + cid
    pltpu.sync_copy(dst_hbm.at[pl.ds(wid * EPW, EPW)], idx_v)

    def zero_body(i, c):
        cnt_v[pl.ds(i * L, L)] = jnp.zeros((L,), jnp.float32)
        return c

    lax.fori_loop(0, N // L, zero_body, 0)

    ones = jnp.ones((L,), jnp.float32)

    def scat_body(i, c):
        idx = idx_v[pl.ds(i * L, L)]
        plsc.addupdate_scatter(cnt_v, [idx], ones)
        return c

    lax.fori_loop(0, EPW // L, scat_body, 0)
    pltpu.sync_copy(cnt_v, out_hbm.at[pl.ds(wid * N, N)])


def _prep_body(x_ref, w_ref, cnt_ref, hs_ref, dinv_ref):
    deg = jnp.sum(cnt_ref[...], axis=0) + 1.0
    dinv = lax.rsqrt(deg)
    h = jnp.dot(x_ref[...], w_ref[...], preferred_element_type=jnp.float32)
    hs_ref[...] = h * dinv[:, None]
    dinv_ref[...] = dinv[:, None]


_prep_kernel = pl.pallas_call(
    _prep_body,
    out_shape=(
        jax.ShapeDtypeStruct((N, D), jnp.float32),
        jax.ShapeDtypeStruct((N, 1), jnp.float32),
    ),
)


@functools.partial(
    pl.kernel,
    out_type=jax.ShapeDtypeStruct((NC, NP, D), jnp.float32),
    mesh=_mesh,
    scratch_types=[
        pltpu.VMEM((SB, CHUNK), jnp.int32),
        pltpu.VMEM((SB, CHUNK), jnp.int32),
        pltpu.VMEM((CHUNK, D), jnp.float32),
        pltpu.VMEM((CHUNK, D), jnp.float32),
        pltpu.SemaphoreType.DMA,
        pltpu.SemaphoreType.DMA,
        pltpu.VMEM_SHARED((NP, D), jnp.float32),
    ],
    compiler_params=_sc_params,
)
def _agg_kernel(hs_hbm, src_hbm, dst_hbm, acc_hbm,
                src_idx, dst_idx, rows0, rows1, sem0, sem1, acc_sh):
    cid = lax.axis_index("c")
    sid = lax.axis_index("s")
    wid = sid * NC + cid

    def zero_rows(i, c):
        rows0[i // (D // L), pl.ds((i % (D // L)) * L, L)] = (
            jnp.zeros((L,), jnp.float32))
        return c

    lax.fori_loop(0, CHUNK * (D // L), zero_rows, 0)

    def zero_acc(j, c):
        pltpu.sync_copy(rows0, acc_sh.at[pl.ds(sid * RPT + j * CHUNK, CHUNK)])
        return c

    lax.fori_loop(0, RPT // CHUNK, zero_acc, 0)
    plsc.subcore_barrier()

    def superblock(b, carry):
        pltpu.sync_copy(src_hbm.at[wid, b], src_idx)
        pltpu.sync_copy(dst_hbm.at[wid, b], dst_idx)
        pltpu.async_copy(hs_hbm.at[src_idx.at[0]], rows0, sem0)

        def step(it, c):
            c0 = it * 2
            pltpu.async_copy(hs_hbm.at[src_idx.at[c0 + 1]], rows1, sem1)
            pltpu.make_async_copy(hs_hbm.at[src_idx.at[c0]], rows0,
                                  sem0).wait()
            pltpu.sync_copy(rows0, acc_sh.at[dst_idx.at[c0]], add=True)
            pltpu.async_copy(hs_hbm.at[src_idx.at[c0 + 2]], rows0, sem0)
            pltpu.make_async_copy(hs_hbm.at[src_idx.at[c0 + 1]], rows1,
                                  sem1).wait()
            pltpu.sync_copy(rows1, acc_sh.at[dst_idx.at[c0 + 1]], add=True)
            return c

        lax.fori_loop(0, (SB - 1) // 2, step, 0)
        pltpu.make_async_copy(hs_hbm.at[src_idx.at[SB - 1]], rows0,
                              sem0).wait()
        pltpu.sync_copy(rows0, acc_sh.at[dst_idx.at[SB - 1]], add=True)
        return carry

    lax.fori_loop(0, NSB, superblock, 0)
    plsc.subcore_barrier()
    pltpu.sync_copy(acc_sh.at[pl.ds(sid * RPT, RPT)],
                    acc_hbm.at[cid, pl.ds(sid * RPT, RPT)])


def _post_body(acc_ref, hs_ref, dinv_ref, b_ref, g_ref, be_ref, out_ref):
    a0 = acc_ref[0, pl.ds(0, N), :]
    a1 = acc_ref[1, pl.ds(0, N), :]
    pre = (a0 + a1 + hs_ref[...]) * dinv_ref[...] + b_ref[...]
    pre = jnp.where(pre >= 0.0, pre, 0.01 * pre)
    mean = jnp.mean(pre, axis=0, keepdims=True)
    var = jnp.mean((pre - mean) ** 2, axis=0, keepdims=True)
    out_ref[...] = (pre - mean) * lax.rsqrt(var + 1e-5) * g_ref[...] + be_ref[...]


_post_kernel = pl.pallas_call(
    _post_body,
    out_shape=jax.ShapeDtypeStruct((N, D), jnp.float32),
)


def kernel(x, edge_index, W, b, gamma, beta):
    src = edge_index[0]
    dst = edge_index[1]
    cnt = _deg_kernel(dst).reshape(NW, N)
    hs, dinv = _prep_kernel(x, W, cnt)
    acc = _agg_kernel(hs,
                      src.reshape(NW, NSB, SB, CHUNK),
                      dst.reshape(NW, NSB, SB, CHUNK))
    return _post_kernel(acc, hs, dinv,
                        b.reshape(1, D), gamma.reshape(1, D),
                        beta.reshape(1, D))


# speedup vs baseline: 42.5228x; 1.0504x over previous
import functools

import jax
import jax.numpy as jnp
from jax import lax
from jax.experimental import pallas as pl
from jax.experimental.pallas import tpu as pltpu
from jax.experimental.pallas import tpu_sc as plsc

N = 10000
E = 320000
D = 128

NC, NS, L = 2, 16, 16
NW = NC * NS
EPW = E // NW
CHUNK = 40
NCHUNK = EPW // CHUNK
NP = 10240
RPT = NP // NS
NBUF = 5
SB = 25
NSB = NCHUNK // SB
NG = SB // NBUF

_mesh = plsc.VectorSubcoreMesh(core_axis_name="c", subcore_axis_name="s")
_sc_params = pltpu.CompilerParams(needs_layout_passes=False)


@functools.partial(
    pl.kernel,
    out_type=jax.ShapeDtypeStruct((NW * N,), jnp.float32),
    mesh=_mesh,
    scratch_types=[
        pltpu.VMEM((EPW,), jnp.int32),
        pltpu.VMEM((N,), jnp.float32),
    ],
    compiler_params=_sc_params,
)
def _deg_kernel(dst_hbm, out_hbm, idx_v, cnt_v):
    cid = lax.axis_index("c")
    sid = lax.axis_index("s")
    wid = sid * NC + cid
    pltpu.sync_copy(dst_hbm.at[pl.ds(wid * EPW, EPW)], idx_v)

    def zero_body(i, c):
        cnt_v[pl.ds(i * L, L)] = jnp.zeros((L,), jnp.float32)
        return c

    lax.fori_loop(0, N // L, zero_body, 0)

    ones = jnp.ones((L,), jnp.float32)

    def scat_body(i, c):
        idx = idx_v[pl.ds(i * L, L)]
        plsc.addupdate_scatter(cnt_v, [idx], ones)
        return c

    lax.fori_loop(0, EPW // L, scat_body, 0)
    pltpu.sync_copy(cnt_v, out_hbm.at[pl.ds(wid * N, N)])


def _prep_body(x_ref, w_ref, cnt_ref, hs_ref, dinv_ref):
    deg = jnp.sum(cnt_ref[...], axis=0) + 1.0
    dinv = lax.rsqrt(deg)
    h = jnp.dot(x_ref[...], w_ref[...], preferred_element_type=jnp.float32)
    hs_ref[...] = h * dinv[:, None]
    dinv_ref[...] = dinv[:, None]


_prep_kernel = pl.pallas_call(
    _prep_body,
    out_shape=(
        jax.ShapeDtypeStruct((N, D), jnp.float32),
        jax.ShapeDtypeStruct((N, 1), jnp.float32),
    ),
)


@functools.partial(
    pl.kernel,
    out_type=jax.ShapeDtypeStruct((NC, NP, D), jnp.float32),
    mesh=_mesh,
    scratch_types=[
        pltpu.VMEM((SB, CHUNK), jnp.int32),
        pltpu.VMEM((SB, CHUNK), jnp.int32),
    ] + [pltpu.VMEM((CHUNK, D), jnp.float32)] * NBUF
      + [pltpu.SemaphoreType.DMA] * NBUF
      + [pltpu.VMEM_SHARED((NP, D), jnp.float32)],
    compiler_params=_sc_params,
)
def _agg_kernel(hs_hbm, src_hbm, dst_hbm, acc_hbm,
                src_idx, dst_idx, r0, r1, r2, r3, r4,
                s0, s1, s2, s3, s4, acc_sh):
    cid = lax.axis_index("c")
    sid = lax.axis_index("s")
    wid = sid * NC + cid
    rows = [r0, r1, r2, r3, r4]
    sems = [s0, s1, s2, s3, s4]

    def zero_rows(i, c):
        r0[i // (D // L), pl.ds((i % (D // L)) * L, L)] = (
            jnp.zeros((L,), jnp.float32))
        return c

    lax.fori_loop(0, CHUNK * (D // L), zero_rows, 0)

    def zero_acc(j, c):
        pltpu.sync_copy(r0, acc_sh.at[pl.ds(sid * RPT + j * CHUNK, CHUNK)])
        return c

    lax.fori_loop(0, RPT // CHUNK, zero_acc, 0)
    plsc.subcore_barrier()

    def superblock(sb, carry):
        pltpu.sync_copy(src_hbm.at[wid, sb], src_idx)
        pltpu.sync_copy(dst_hbm.at[wid, sb], dst_idx)
        for b in range(NBUF):
            pltpu.async_copy(hs_hbm.at[src_idx.at[b]], rows[b], sems[b])

        def turn(g, c):
            c0 = g * NBUF
            for b in range(NBUF):
                pltpu.make_async_copy(hs_hbm.at[src_idx.at[c0 + b]], rows[b],
                                      sems[b]).wait()
                pltpu.sync_copy(rows[b], acc_sh.at[dst_idx.at[c0 + b]],
                                add=True)

                @pl.when(g < NG - 1)
                def _():
                    pltpu.async_copy(hs_hbm.at[src_idx.at[c0 + b + NBUF]],
                                     rows[b], sems[b])

            return c

        lax.fori_loop(0, NG, turn, 0)
        return carry

    lax.fori_loop(0, NSB, superblock, 0)
    plsc.subcore_barrier()
    pltpu.sync_copy(acc_sh.at[pl.ds(sid * RPT, RPT)],
                    acc_hbm.at[cid, pl.ds(sid * RPT, RPT)])


def _post_body(acc_ref, hs_ref, dinv_ref, b_ref, g_ref, be_ref, out_ref):
    a0 = acc_ref[0, pl.ds(0, N), :]
    a1 = acc_ref[1, pl.ds(0, N), :]
    pre = (a0 + a1 + hs_ref[...]) * dinv_ref[...] + b_ref[...]
    pre = jnp.where(pre >= 0.0, pre, 0.01 * pre)
    mean = jnp.mean(pre, axis=0, keepdims=True)
    var = jnp.mean((pre - mean) ** 2, axis=0, keepdims=True)
    out_ref[...] = (pre - mean) * lax.rsqrt(var + 1e-5) * g_ref[...] + be_ref[...]


_post_kernel = pl.pallas_call(
    _post_body,
    out_shape=jax.ShapeDtypeStruct((N, D), jnp.float32),
)


def kernel(x, edge_index, W, b, gamma, beta):
    src = edge_index[0]
    dst = edge_index[1]
    cnt = _deg_kernel(dst).reshape(NW, N)
    hs, dinv = _prep_kernel(x, W, cnt)
    acc = _agg_kernel(hs,
                      src.reshape(NW, NSB, SB, CHUNK),
                      dst.reshape(NW, NSB, SB, CHUNK))
    return _post_kernel(acc, hs, dinv,
                        b.reshape(1, D), gamma.reshape(1, D),
                        beta.reshape(1, D))


# speedup vs baseline: 45.9469x; 1.0805x over previous
import functools

import jax
import jax.numpy as jnp
from jax import lax
from jax.experimental import pallas as pl
from jax.experimental.pallas import tpu as pltpu
from jax.experimental.pallas import tpu_sc as plsc

N = 10000
E = 320000
D = 128

NC, NS, L = 2, 16, 16
NW = NC * NS
EPW = E // NW
CHUNK = 40
NCHUNK = EPW // CHUNK
NP = 10240
RPT = NP // NS
NBUF = 5
SB = 50
NSB = NCHUNK // SB
NG = SB // NBUF

_mesh = plsc.VectorSubcoreMesh(core_axis_name="c", subcore_axis_name="s")
_sc_params = pltpu.CompilerParams(needs_layout_passes=False)


@functools.partial(
    pl.kernel,
    out_type=jax.ShapeDtypeStruct((NW * N,), jnp.float32),
    mesh=_mesh,
    scratch_types=[
        pltpu.VMEM((EPW,), jnp.int32),
        pltpu.VMEM((N,), jnp.float32),
    ],
    compiler_params=_sc_params,
)
def _deg_kernel(dst_hbm, out_hbm, idx_v, cnt_v):
    cid = lax.axis_index("c")
    sid = lax.axis_index("s")
    wid = sid * NC + cid
    pltpu.sync_copy(dst_hbm.at[pl.ds(wid * EPW, EPW)], idx_v)

    def zero_body(i, c):
        cnt_v[pl.ds(i * L, L)] = jnp.zeros((L,), jnp.float32)
        return c

    lax.fori_loop(0, N // L, zero_body, 0)

    ones = jnp.ones((L,), jnp.float32)

    def scat_body(i, c):
        idx = idx_v[pl.ds(i * L, L)]
        plsc.addupdate_scatter(cnt_v, [idx], ones)
        return c

    lax.fori_loop(0, EPW // L, scat_body, 0)
    pltpu.sync_copy(cnt_v, out_hbm.at[pl.ds(wid * N, N)])


def _prep_body(x_ref, w_ref, cnt_ref, hs_ref, dinv_ref):
    deg = jnp.sum(cnt_ref[...], axis=0) + 1.0
    dinv = lax.rsqrt(deg)
    h = jnp.dot(x_ref[...], w_ref[...], preferred_element_type=jnp.float32)
    hs_ref[...] = h * dinv[:, None]
    dinv_ref[...] = dinv[:, None]


_prep_kernel = pl.pallas_call(
    _prep_body,
    out_shape=(
        jax.ShapeDtypeStruct((N, D), jnp.float32),
        jax.ShapeDtypeStruct((N, 1), jnp.float32),
    ),
)


@functools.partial(
    pl.kernel,
    out_type=jax.ShapeDtypeStruct((NC, NP, D), jnp.float32),
    mesh=_mesh,
    scratch_types=[
        pltpu.VMEM((SB, CHUNK), jnp.int32),
        pltpu.VMEM((SB, CHUNK), jnp.int32),
    ] + [pltpu.VMEM((CHUNK, D), jnp.float32)] * NBUF
      + [pltpu.SemaphoreType.DMA] * NBUF
      + [pltpu.VMEM_SHARED((NP, D), jnp.float32)],
    compiler_params=_sc_params,
)
def _agg_kernel(hs_hbm, src_hbm, dst_hbm, acc_hbm,
                src_idx, dst_idx, r0, r1, r2, r3, r4,
                s0, s1, s2, s3, s4, acc_sh):
    cid = lax.axis_index("c")
    sid = lax.axis_index("s")
    wid = sid * NC + cid
    rows = [r0, r1, r2, r3, r4]
    sems = [s0, s1, s2, s3, s4]

    def zero_rows(i, c):
        r0[i // (D // L), pl.ds((i % (D // L)) * L, L)] = (
            jnp.zeros((L,), jnp.float32))
        return c

    lax.fori_loop(0, CHUNK * (D // L), zero_rows, 0)

    def zero_acc(j, c):
        pltpu.sync_copy(r0, acc_sh.at[pl.ds(sid * RPT + j * CHUNK, CHUNK)])
        return c

    lax.fori_loop(0, RPT // CHUNK, zero_acc, 0)
    plsc.subcore_barrier()

    def superblock(sb, carry):
        pltpu.sync_copy(src_hbm.at[wid, sb], src_idx)
        pltpu.sync_copy(dst_hbm.at[wid, sb], dst_idx)
        for b in range(NBUF):
            pltpu.async_copy(hs_hbm.at[src_idx.at[b]], rows[b], sems[b])

        def turn(g, c):
            c0 = g * NBUF
            for b in range(NBUF):
                pltpu.make_async_copy(hs_hbm.at[src_idx.at[c0 + b]], rows[b],
                                      sems[b]).wait()
                pltpu.sync_copy(rows[b], acc_sh.at[dst_idx.at[c0 + b]],
                                add=True)

                @pl.when(g < NG - 1)
                def _():
                    pltpu.async_copy(hs_hbm.at[src_idx.at[c0 + b + NBUF]],
                                     rows[b], sems[b])

            return c

        lax.fori_loop(0, NG, turn, 0)
        return carry

    lax.fori_loop(0, NSB, superblock, 0)
    plsc.subcore_barrier()
    pltpu.sync_copy(acc_sh.at[pl.ds(sid * RPT, RPT)],
                    acc_hbm.at[cid, pl.ds(sid * RPT, RPT)])


def _post_body(acc_ref, hs_ref, dinv_ref, b_ref, g_ref, be_ref, out_ref):
    a0 = acc_ref[0, pl.ds(0, N), :]
    a1 = acc_ref[1, pl.ds(0, N), :]
    pre = (a0 + a1 + hs_ref[...]) * dinv_ref[...] + b_ref[...]
    pre = jnp.where(pre >= 0.0, pre, 0.01 * pre)
    mean = jnp.mean(pre, axis=0, keepdims=True)
    var = jnp.mean((pre - mean) ** 2, axis=0, keepdims=True)
    out_ref[...] = (pre - mean) * lax.rsqrt(var + 1e-5) * g_ref[...] + be_ref[...]


_post_kernel = pl.pallas_call(
    _post_body,
    out_shape=jax.ShapeDtypeStruct((N, D), jnp.float32),
)


def kernel(x, edge_index, W, b, gamma, beta):
    src = edge_index[0]
    dst = edge_index[1]
    cnt = _deg_kernel(dst).reshape(NW, N)
    hs, dinv = _prep_kernel(x, W, cnt)
    acc = _agg_kernel(hs,
                      src.reshape(NW, NSB, SB, CHUNK),
                      dst.reshape(NW, NSB, SB, CHUNK))
    return _post_kernel(acc, hs, dinv,
                        b.reshape(1, D), gamma.reshape(1, D),
                        beta.reshape(1, D))
